# baseline (speedup 1.0000x reference)
import jax
import jax.numpy as jnp
from jax.experimental import pallas as pl
from jax.experimental.pallas import tpu as pltpu

_N_MEM = 1000000
_BLK = 4000
_NB = _N_MEM // _BLK
_ROWS = 64
_COLS = _N_MEM // _ROWS
_COLSP = 15744
_NA = 16
_EPS = 1e-12
_NEG = float("-inf")
_BIGI = 2 ** 30


def _phase1_body(z_ref, embs_ref, act_ref, rew_ref, sims_ref, num_ref, cnt_ref):
    i = pl.program_id(0)
    z = z_ref[...]
    zn = z / jnp.maximum(jnp.sqrt(jnp.sum(z * z)), _EPS)
    e = embs_ref[...]
    dims_col = (((1,), (0,)), ((), ()))
    ee = e * e
    eeh = ee.astype(jnp.bfloat16)
    eel = (ee - eeh.astype(jnp.float32)).astype(jnp.bfloat16)
    ones_col = jnp.ones((64, 1), jnp.bfloat16)
    sq = (jax.lax.dot_general(eeh, ones_col, dims_col,
                              preferred_element_type=jnp.float32)
          + jax.lax.dot_general(eel, ones_col, dims_col,
                                preferred_element_type=jnp.float32))
    en = e / jnp.maximum(jnp.sqrt(sq), _EPS)
    dims_row = (((1,), (1,)), ((), ()))
    s = jax.lax.dot_general(zn.astype(jnp.bfloat16), en.astype(jnp.bfloat16),
                            dims_row, preferred_element_type=jnp.float32)
    sims_ref[0] = s

    a = act_ref[0]
    r = rew_ref[0]
    aid = jax.lax.broadcasted_iota(jnp.int32, (_NA, _BLK), 0)
    mask = aid == a
    numv = jnp.sum(jnp.where(mask, s * r, 0.0), axis=1, keepdims=True)
    cntv = jnp.sum(jnp.where(mask, jnp.abs(s), 0.0), axis=1, keepdims=True)

    @pl.when(i == 0)
    def _init():
        num_ref[...] = numv
        cnt_ref[...] = cntv

    @pl.when(i > 0)
    def _acc():
        num_ref[...] += numv
        cnt_ref[...] += cntv


def _phase2_body(sims_ref, num_ref, cnt_ref, wm_ref,
                 tv_ref, ti_ref, q_ref, sc_ref, pr_ref, s_scr):
    s_scr[...] = sims_ref[...]
    col_iota = jax.lax.broadcasted_iota(jnp.int32, (_ROWS, _COLSP), 1)
    lane64 = jax.lax.broadcasted_iota(jnp.int32, (_ROWS, 64), 1)
    row64 = jax.lax.broadcasted_iota(jnp.int32, (_ROWS, 64), 0)

    def ext_body(i, carry):
        V, C = carry
        S = s_scr[...]
        m = jnp.max(S, axis=1, keepdims=True)
        cidx = jnp.min(jnp.where(S == m, col_iota, _BIGI),
                       axis=1, keepdims=True)
        s_scr[...] = jnp.where(col_iota == cidx, _NEG, S)
        V = jnp.where(lane64 == i, m, V)
        C = jnp.where(lane64 == i, cidx, C)
        return V, C

    V0 = jnp.full((_ROWS, 64), _NEG, jnp.float32)
    C0 = jnp.zeros((_ROWS, 64), jnp.int32)
    V, C = jax.lax.fori_loop(0, 64, ext_body, (V0, C0))
    G = row64 * _COLS + C

    lane = jax.lax.broadcasted_iota(jnp.int32, (1, 64), 1)

    def mrg_body(j, carry):
        V2, outv, outi = carry
        mv = jnp.max(V2)
        g = jnp.min(jnp.where(V2 == mv, G, _BIGI))
        outv = jnp.where(lane == j, mv, outv)
        outi = jnp.where(lane == j, g, outi)
        V2 = jnp.where(G == g, _NEG, V2)
        return V2, outv, outi

    _, outv, outi = jax.lax.fori_loop(
        0, 64, mrg_body,
        (V, jnp.zeros((1, 64), jnp.float32), jnp.zeros((1, 64), jnp.int32)))
    tv_ref[...] = outv
    ti_ref[...] = outi

    num = num_ref[...]
    cnt = cnt_ref[...] + 1e-6
    q = num / cnt
    sc = wm_ref[0, 0] * q
    mx = jnp.max(sc)
    p = jnp.exp(sc - mx)
    pr = p / jnp.sum(p)
    q_ref[...] = q
    sc_ref[...] = sc
    pr_ref[...] = pr


def kernel(z, embs, actions, rewards, k, w_reward, w_memory):
    z2 = jnp.reshape(z, (1, 64)).astype(jnp.float32)
    e2 = embs.astype(jnp.float32)
    a3 = jnp.reshape(actions, (_NB, 1, _BLK)).astype(jnp.int32)
    r3 = jnp.reshape(rewards, (_NB, 1, _BLK))

    sims, num, cnt = pl.pallas_call(
        _phase1_body,
        grid=(_NB,),
        in_specs=[
            pl.BlockSpec((1, 64), lambda i: (0, 0)),
            pl.BlockSpec((_BLK, 64), lambda i: (i, 0)),
            pl.BlockSpec((1, 1, _BLK), lambda i: (i, 0, 0)),
            pl.BlockSpec((1, 1, _BLK), lambda i: (i, 0, 0)),
        ],
        out_specs=[
            pl.BlockSpec((1, 1, _BLK), lambda i: (i, 0, 0)),
            pl.BlockSpec((_NA, 1), lambda i: (0, 0)),
            pl.BlockSpec((_NA, 1), lambda i: (0, 0)),
        ],
        out_shape=[
            jax.ShapeDtypeStruct((_NB, 1, _BLK), jnp.float32),
            jax.ShapeDtypeStruct((_NA, 1), jnp.float32),
            jax.ShapeDtypeStruct((_NA, 1), jnp.float32),
        ],
        compiler_params=pltpu.CompilerParams(
            dimension_semantics=("arbitrary",)),
    )(z2, e2, a3, r3)

    sims2 = jnp.pad(jnp.reshape(sims, (_ROWS, _COLS)),
                    ((0, 0), (0, _COLSP - _COLS)),
                    constant_values=float("-inf"))
    wm = jnp.reshape(jnp.asarray(w_memory, jnp.float32), (1, 1))

    tv, ti, q, sc, pr = pl.pallas_call(
        _phase2_body,
        out_shape=[
            jax.ShapeDtypeStruct((1, 64), jnp.float32),
            jax.ShapeDtypeStruct((1, 64), jnp.int32),
            jax.ShapeDtypeStruct((_NA, 1), jnp.float32),
            jax.ShapeDtypeStruct((_NA, 1), jnp.float32),
            jax.ShapeDtypeStruct((_NA, 1), jnp.float32),
        ],
        scratch_shapes=[pltpu.VMEM((_ROWS, _COLSP), jnp.float32)],
    )(sims2, num, cnt, wm)

    return (jnp.reshape(tv, (64,)), jnp.reshape(ti, (64,)),
            jnp.reshape(q, (_NA,)), jnp.reshape(sc, (_NA,)),
            jnp.reshape(pr, (_NA,)))


# speedup vs baseline: 1.2163x; 1.2163x over previous
import jax
import jax.numpy as jnp
from jax.experimental import pallas as pl
from jax.experimental.pallas import tpu as pltpu

_N_MEM = 1000000
_BLK = 4000
_NB = _N_MEM // _BLK
_NA = 16
_EPS = 1e-12
_NEG = float("-inf")
_BIGI = 2 ** 30


def _phase1_body(z_ref, embs_ref, act_ref, rew_ref, sims_ref, num_ref, cnt_ref):
    i = pl.program_id(0)
    z = z_ref[...]
    zn = z / jnp.maximum(jnp.sqrt(jnp.sum(z * z)), _EPS)
    e = embs_ref[0]
    dims_col = (((1,), (0,)), ((), ()))
    ee = e * e
    eeh = ee.astype(jnp.bfloat16)
    eel = (ee - eeh.astype(jnp.float32)).astype(jnp.bfloat16)
    ones_col = jnp.ones((64, 1), jnp.bfloat16)
    sq = (jax.lax.dot_general(eeh, ones_col, dims_col,
                              preferred_element_type=jnp.float32)
          + jax.lax.dot_general(eel, ones_col, dims_col,
                                preferred_element_type=jnp.float32))
    en = e / jnp.maximum(jnp.sqrt(sq), _EPS)
    dims_row = (((1,), (1,)), ((), ()))
    s = jax.lax.dot_general(zn.astype(jnp.bfloat16), en.astype(jnp.bfloat16),
                            dims_row, preferred_element_type=jnp.float32)
    sims_ref[0] = s

    a = act_ref[0]
    r = rew_ref[0]
    aid = jax.lax.broadcasted_iota(jnp.int32, (_NA, _BLK), 0)
    mask = aid == a
    numv = jnp.sum(jnp.where(mask, s * r, 0.0), axis=1, keepdims=True)
    cntv = jnp.sum(jnp.where(mask, jnp.abs(s), 0.0), axis=1, keepdims=True)

    @pl.when(i == 0)
    def _init():
        num_ref[...] = numv
        cnt_ref[...] = cntv

    @pl.when(i > 0)
    def _acc():
        num_ref[...] += numv
        cnt_ref[...] += cntv


def _phase2_body(sims_ref, num_ref, cnt_ref, wm_ref,
                 tv_ref, ti_ref, q_ref, sc_ref, pr_ref, s_scr):
    s_scr[...] = jnp.reshape(sims_ref[...], (_NB, _BLK))
    col_iota = jax.lax.broadcasted_iota(jnp.int32, (_NB, _BLK), 1)
    lane64 = jax.lax.broadcasted_iota(jnp.int32, (_NB, 64), 1)
    row64 = jax.lax.broadcasted_iota(jnp.int32, (_NB, 64), 0)

    def ext_body(i, carry):
        V, C = carry
        S = s_scr[...]
        m = jnp.max(S, axis=1, keepdims=True)
        cidx = jnp.min(jnp.where(S == m, col_iota, _BIGI),
                       axis=1, keepdims=True)
        s_scr[...] = jnp.where(col_iota == cidx, _NEG, S)
        V = jnp.where(lane64 == i, m, V)
        C = jnp.where(lane64 == i, cidx, C)
        return V, C

    V0 = jnp.full((_NB, 64), _NEG, jnp.float32)
    C0 = jnp.zeros((_NB, 64), jnp.int32)
    V, C = jax.lax.fori_loop(0, 64, ext_body, (V0, C0))
    G = row64 * _BLK + C

    lane = jax.lax.broadcasted_iota(jnp.int32, (1, 64), 1)

    def mrg_body(j, carry):
        V2, outv, outi = carry
        mv = jnp.max(V2)
        g = jnp.min(jnp.where(V2 == mv, G, _BIGI))
        outv = jnp.where(lane == j, mv, outv)
        outi = jnp.where(lane == j, g, outi)
        V2 = jnp.where(G == g, _NEG, V2)
        return V2, outv, outi

    _, outv, outi = jax.lax.fori_loop(
        0, 64, mrg_body,
        (V, jnp.zeros((1, 64), jnp.float32), jnp.zeros((1, 64), jnp.int32)))
    tv_ref[...] = outv
    ti_ref[...] = outi

    num = num_ref[...]
    cnt = cnt_ref[...] + 1e-6
    q = num / cnt
    sc = wm_ref[0, 0] * q
    mx = jnp.max(sc)
    p = jnp.exp(sc - mx)
    pr = p / jnp.sum(p)
    q_ref[...] = q
    sc_ref[...] = sc
    pr_ref[...] = pr


def kernel(z, embs, actions, rewards, k, w_reward, w_memory):
    z2 = jnp.reshape(z, (1, 64)).astype(jnp.float32)
    e3 = jnp.reshape(embs, (_NB, _BLK, 64))
    a3 = jnp.reshape(actions, (_NB, 1, _BLK)).astype(jnp.int32)
    r3 = jnp.reshape(rewards, (_NB, 1, _BLK))

    sims, num, cnt = pl.pallas_call(
        _phase1_body,
        grid=(_NB,),
        in_specs=[
            pl.BlockSpec((1, 64), lambda i: (0, 0)),
            pl.BlockSpec((1, _BLK, 64), lambda i: (i, 0, 0)),
            pl.BlockSpec((1, 1, _BLK), lambda i: (i, 0, 0)),
            pl.BlockSpec((1, 1, _BLK), lambda i: (i, 0, 0)),
        ],
        out_specs=[
            pl.BlockSpec((1, 1, _BLK), lambda i: (i, 0, 0)),
            pl.BlockSpec((_NA, 1), lambda i: (0, 0)),
            pl.BlockSpec((_NA, 1), lambda i: (0, 0)),
        ],
        out_shape=[
            jax.ShapeDtypeStruct((_NB, 1, _BLK), jnp.float32),
            jax.ShapeDtypeStruct((_NA, 1), jnp.float32),
            jax.ShapeDtypeStruct((_NA, 1), jnp.float32),
        ],
        compiler_params=pltpu.CompilerParams(
            dimension_semantics=("arbitrary",)),
    )(z2, e3, a3, r3)

    wm = jnp.reshape(jnp.asarray(w_memory, jnp.float32), (1, 1))

    tv, ti, q, sc, pr = pl.pallas_call(
        _phase2_body,
        out_shape=[
            jax.ShapeDtypeStruct((1, 64), jnp.float32),
            jax.ShapeDtypeStruct((1, 64), jnp.int32),
            jax.ShapeDtypeStruct((_NA, 1), jnp.float32),
            jax.ShapeDtypeStruct((_NA, 1), jnp.float32),
            jax.ShapeDtypeStruct((_NA, 1), jnp.float32),
        ],
        scratch_shapes=[pltpu.VMEM((_NB, _BLK), jnp.float32)],
    )(sims, num, cnt, wm)

    return (jnp.reshape(tv, (64,)), jnp.reshape(ti, (64,)),
            jnp.reshape(q, (_NA,)), jnp.reshape(sc, (_NA,)),
            jnp.reshape(pr, (_NA,)))
